# baseline (speedup 1.0000x reference)
import functools

import jax
import jax.numpy as jnp
from jax import lax
from jax.experimental import pallas as pl
from jax.experimental.pallas import tpu as pltpu
from jax.experimental.pallas import tpu_sc as plsc


def _means_body(m_ref, md_ref, mo_ref):
    mo_ref[...] = m_ref[...] + md_ref[...]


_N = 1000000
_W = 7936
_NSLAB = _N // _W
_TAIL = _N - _NSLAB * _W
_NWORKERS = 32
_NITER = (_NSLAB + _NWORKERS) // _NWORKERS


def _rot_sc_kernel(n):
    mesh = plsc.VectorSubcoreMesh(core_axis_name="c", subcore_axis_name="s")

    @functools.partial(
        pl.kernel,
        mesh=mesh,
        out_type=jax.ShapeDtypeStruct((4, n), jnp.float32),
        scratch_types=[
            pltpu.VMEM((4, _W), jnp.float32),
            pltpu.VMEM((4, _W), jnp.float32),
            pltpu.VMEM((4, _TAIL), jnp.float32),
            pltpu.VMEM((4, _TAIL), jnp.float32),
        ],
    )
    def rot_add(r_hbm, rd_hbm, out_hbm, a_v, b_v, at_v, bt_v):
        wid = lax.axis_index("s") * 2 + lax.axis_index("c")

        def add_rows(a, b, w):
            def body(i, _):
                for r in range(4):
                    sl = pl.ds(i * 16, 16)
                    a[r, sl] = a[r, sl] + b[r, sl]
                return 0

            lax.fori_loop(0, w // 16, body, 0)

        def do_slab(g):
            col = g * _W
            pltpu.sync_copy(r_hbm.at[:, pl.ds(col, _W)], a_v)
            pltpu.sync_copy(rd_hbm.at[:, pl.ds(col, _W)], b_v)
            add_rows(a_v, b_v, _W)
            pltpu.sync_copy(a_v, out_hbm.at[:, pl.ds(col, _W)])

        for it in range(_NITER):
            g = it * _NWORKERS + wid

            @pl.when(g < _NSLAB)
            def _():
                do_slab(g)

        @pl.when(wid == _NWORKERS - 1)
        def _():
            col = _NSLAB * _W
            pltpu.sync_copy(r_hbm.at[:, pl.ds(col, _TAIL)], at_v)
            pltpu.sync_copy(rd_hbm.at[:, pl.ds(col, _TAIL)], bt_v)
            add_rows(at_v, bt_v, _TAIL)
            pltpu.sync_copy(at_v, out_hbm.at[:, pl.ds(col, _TAIL)])

    return rot_add


def kernel(means, scales, rot, means_def, rot_def):
    n = means.shape[0]
    B = 32768
    g = pl.cdiv(n, B)
    bs3 = pl.BlockSpec((3, B), lambda i: (0, i))
    mo_t = pl.pallas_call(
        _means_body,
        grid=(g,),
        in_specs=[bs3, bs3],
        out_specs=bs3,
        out_shape=jax.ShapeDtypeStruct((3, n), means.dtype),
    )(means.T, means_def.T)
    ro_t = _rot_sc_kernel(n)(rot.T, rot_def.T)
    return (mo_t.T, scales, ro_t.T)


# speedup vs baseline: 1.0089x; 1.0089x over previous
import functools

import jax
import jax.numpy as jnp
from jax import lax
from jax.experimental import pallas as pl
from jax.experimental.pallas import tpu as pltpu
from jax.experimental.pallas import tpu_sc as plsc


def _means_body(m_ref, md_ref, mo_ref):
    mo_ref[...] = m_ref[...] + md_ref[...]


_N = 1000000
_W = 5376
_S = _N // _W
_COVER = _S * _W
_NW = 32
_NIT = (_S + _NW - 1) // _NW


def _rot_sc_kernel(n):
    mesh = plsc.VectorSubcoreMesh(core_axis_name="c", subcore_axis_name="s")

    @functools.partial(
        pl.kernel,
        mesh=mesh,
        out_type=jax.ShapeDtypeStruct((4, n), jnp.float32),
        scratch_types=(
            [pltpu.VMEM((4, _W), jnp.float32) for _ in range(6)]
            + [pltpu.SemaphoreType.DMA for _ in range(9)]
        ),
    )
    def rot_add(r_hbm, rd_hbm, out_hbm, *scratch):
        a = scratch[0:3]
        b = scratch[3:6]
        sa = scratch[6:9]
        sb = scratch[9:12]
        so = scratch[12:15]
        wid = lax.axis_index("s") * 2 + lax.axis_index("c")

        def in_copies(it):
            k = it % 3
            g = it * _NW + wid
            return g, [
                (r_hbm.at[:, pl.ds(g * _W, _W)], a[k], sa[k]),
                (rd_hbm.at[:, pl.ds(g * _W, _W)], b[k], sb[k]),
            ]

        def start_in(it):
            g, pairs = in_copies(it)

            @pl.when(g < _S)
            def _():
                for src, dst, sem in pairs:
                    pltpu.make_async_copy(src, dst, sem).start()

        def wait_in(it):
            g, pairs = in_copies(it)

            @pl.when(g < _S)
            def _():
                for src, dst, sem in pairs:
                    pltpu.make_async_copy(src, dst, sem).wait()

        def compute(it):
            k = it % 3
            g = it * _NW + wid

            @pl.when(g < _S)
            def _():
                @plsc.parallel_loop(0, _W, 16, unroll=4)
                def _(i):
                    sl = pl.ds(i, 16)
                    for r in range(4):
                        a[k][r, sl] = a[k][r, sl] + b[k][r, sl]

        def out_args(it):
            k = it % 3
            g = it * _NW + wid
            return g, (a[k], out_hbm.at[:, pl.ds(g * _W, _W)], so[k])

        def start_out(it):
            g, args = out_args(it)

            @pl.when(g < _S)
            def _():
                pltpu.make_async_copy(*args).start()

        def wait_out(it):
            g, args = out_args(it)

            @pl.when(g < _S)
            def _():
                pltpu.make_async_copy(*args).wait()

        start_in(0)
        start_in(1)
        for it in range(_NIT):
            wait_in(it)
            compute(it)
            start_out(it)
            nxt = it + 2
            if nxt < _NIT:
                if it - 1 >= 0:
                    wait_out(it - 1)
                start_in(nxt)
        for it in range(max(_NIT - 3, 0), _NIT):
            wait_out(it)

    return rot_add


def kernel(means, scales, rot, means_def, rot_def):
    n = means.shape[0]
    B = 32768
    g = pl.cdiv(n, B)
    bs3 = pl.BlockSpec((3, B), lambda i: (0, i))
    mo_t = pl.pallas_call(
        _means_body,
        grid=(g,),
        in_specs=[bs3, bs3],
        out_specs=bs3,
        out_shape=jax.ShapeDtypeStruct((3, n), means.dtype),
    )(means.T, means_def.T)
    ro_t = _rot_sc_kernel(n)(rot.T, rot_def.T)
    tail = rot[_COVER:] + rot_def[_COVER:]
    ro = lax.dynamic_update_slice(ro_t.T, tail, (_COVER, 0))
    return (mo_t.T, scales, ro)


# speedup vs baseline: 1.0837x; 1.0741x over previous
import functools

import jax
import jax.numpy as jnp
from jax import lax
from jax.experimental import pallas as pl
from jax.experimental.pallas import tpu as pltpu
from jax.experimental.pallas import tpu_sc as plsc


def _add_body(m_ref, md_ref, r_ref, rd_ref, mo_ref, ro_ref):
    mo_ref[...] = m_ref[...] + md_ref[...]
    ro_ref[...] = r_ref[...] + rd_ref[...]


_N = 1000000
_W = 5376
_S = _N // _W
_COVER = _S * _W
_NW = 32
_NIT = (_S + _NW - 1) // _NW
_RING = 3


def _scales_sc_copy(n):
    mesh = plsc.VectorSubcoreMesh(core_axis_name="c", subcore_axis_name="s")

    @functools.partial(
        pl.kernel,
        mesh=mesh,
        out_type=jax.ShapeDtypeStruct((3, n), jnp.float32),
        scratch_types=(
            [pltpu.VMEM((3, _W), jnp.float32) for _ in range(_RING)]
            + [pltpu.SemaphoreType.DMA for _ in range(2 * _RING)]
        ),
    )
    def scopy(s_hbm, out_hbm, *scratch):
        buf = scratch[0:_RING]
        si = scratch[_RING : 2 * _RING]
        so = scratch[2 * _RING : 3 * _RING]
        wid = lax.axis_index("s") * 2 + lax.axis_index("c")

        def in_args(it):
            k = it % _RING
            g = it * _NW + wid
            return g, (s_hbm.at[:, pl.ds(g * _W, _W)], buf[k], si[k])

        def out_args(it):
            k = it % _RING
            g = it * _NW + wid
            return g, (buf[k], out_hbm.at[:, pl.ds(g * _W, _W)], so[k])

        def guarded(it, args_fn, method):
            g, args = args_fn(it)

            @pl.when(g < _S)
            def _():
                getattr(pltpu.make_async_copy(*args), method)()

        guarded(0, in_args, "start")
        guarded(1, in_args, "start")
        for it in range(_NIT):
            guarded(it, in_args, "wait")
            guarded(it, out_args, "start")
            nxt = it + 2
            if nxt < _NIT:
                if it - 1 >= 0:
                    guarded(it - 1, out_args, "wait")
                guarded(nxt, in_args, "start")
        for it in range(max(_NIT - 3, 0), _NIT):
            guarded(it, out_args, "wait")

    return scopy


def kernel(means, scales, rot, means_def, rot_def):
    n = means.shape[0]
    B = 32768
    g = pl.cdiv(n, B)
    bs3 = pl.BlockSpec((3, B), lambda i: (0, i))
    bs4 = pl.BlockSpec((4, B), lambda i: (0, i))
    mo_t, ro_t = pl.pallas_call(
        _add_body,
        grid=(g,),
        in_specs=[bs3, bs3, bs4, bs4],
        out_specs=[bs3, bs4],
        out_shape=[
            jax.ShapeDtypeStruct((3, n), means.dtype),
            jax.ShapeDtypeStruct((4, n), rot.dtype),
        ],
    )(means.T, means_def.T, rot.T, rot_def.T)
    so_t = _scales_sc_copy(n)(scales.T)
    so = lax.dynamic_update_slice(so_t.T, scales[_COVER:], (_COVER, 0))
    return (mo_t.T, so, ro_t.T)


# speedup vs baseline: 1.3871x; 1.2800x over previous
import jax
import jax.numpy as jnp
from jax.experimental import pallas as pl
from jax.experimental.pallas import tpu as pltpu


def _body(m_ref, md_ref, r_ref, rd_ref, s_ref, mo_ref, ro_ref, so_ref):
    mo_ref[...] = m_ref[...] + md_ref[...]
    ro_ref[...] = r_ref[...] + rd_ref[...]
    so_ref[...] = s_ref[...]


def kernel(means, scales, rot, means_def, rot_def):
    n = means.shape[0]
    B = 32768
    g = pl.cdiv(n, B)
    bs3 = pl.BlockSpec((3, B), lambda i: (0, i))
    bs4 = pl.BlockSpec((4, B), lambda i: (0, i))
    mo_t, ro_t, so_t = pl.pallas_call(
        _body,
        grid=(g,),
        in_specs=[bs3, bs3, bs4, bs4, bs3],
        out_specs=[bs3, bs4, bs3],
        out_shape=[
            jax.ShapeDtypeStruct((3, n), means.dtype),
            jax.ShapeDtypeStruct((4, n), rot.dtype),
            jax.ShapeDtypeStruct((3, n), scales.dtype),
        ],
        compiler_params=pltpu.CompilerParams(vmem_limit_bytes=16 * 1024 * 1024),
    )(means.T, means_def.T, rot.T, rot_def.T, scales.T)
    return (mo_t.T, so_t.T, ro_t.T)


# speedup vs baseline: 1.5349x; 1.1065x over previous
import jax
import jax.numpy as jnp
from jax.experimental import pallas as pl
from jax.experimental.pallas import tpu as pltpu


def _body(m_ref, md_ref, r_ref, rd_ref, s_ref, mo_ref, ro_ref, so_ref):
    mo_ref[...] = m_ref[...] + md_ref[...]
    ro_ref[...] = r_ref[...] + rd_ref[...]
    so_ref[...] = s_ref[...]


def kernel(means, scales, rot, means_def, rot_def):
    n = means.shape[0]
    B = 32768
    g = pl.cdiv(n, B)
    bs3 = pl.BlockSpec((3, B), lambda i: (0, i))
    bs4 = pl.BlockSpec((4, B), lambda i: (0, i))
    mo_t, ro_t, so_t = pl.pallas_call(
        _body,
        grid=(g,),
        in_specs=[bs3, bs3, bs4, bs4, bs3],
        out_specs=[bs3, bs4, bs3],
        out_shape=[
            jax.ShapeDtypeStruct((3, n), means.dtype),
            jax.ShapeDtypeStruct((4, n), rot.dtype),
            jax.ShapeDtypeStruct((3, n), scales.dtype),
        ],
        compiler_params=pltpu.CompilerParams(vmem_limit_bytes=56 * 1024 * 1024),
    )(means.T, means_def.T, rot.T, rot_def.T, scales.T)
    return (mo_t.T, so_t.T, ro_t.T)


# speedup vs baseline: 1.6707x; 1.0884x over previous
import jax
import jax.numpy as jnp
from jax.experimental import pallas as pl
from jax.experimental.pallas import tpu as pltpu


def _body(m_ref, md_ref, r_ref, rd_ref, s_ref, mo_ref, ro_ref, so_ref):
    mo_ref[...] = m_ref[...] + md_ref[...]
    ro_ref[...] = r_ref[...] + rd_ref[...]
    so_ref[...] = s_ref[...]


def kernel(means, scales, rot, means_def, rot_def):
    n = means.shape[0]
    B = 65536
    g = pl.cdiv(n, B)
    bs3 = pl.BlockSpec((3, B), lambda i: (0, i))
    bs4 = pl.BlockSpec((4, B), lambda i: (0, i))
    mo_t, ro_t, so_t = pl.pallas_call(
        _body,
        grid=(g,),
        in_specs=[bs3, bs3, bs4, bs4, bs3],
        out_specs=[bs3, bs4, bs3],
        out_shape=[
            jax.ShapeDtypeStruct((3, n), means.dtype),
            jax.ShapeDtypeStruct((4, n), rot.dtype),
            jax.ShapeDtypeStruct((3, n), scales.dtype),
        ],
        compiler_params=pltpu.CompilerParams(vmem_limit_bytes=56 * 1024 * 1024),
    )(means.T, means_def.T, rot.T, rot_def.T, scales.T)
    return (mo_t.T, so_t.T, ro_t.T)


# speedup vs baseline: 1.7068x; 1.0216x over previous
import jax
import jax.numpy as jnp
from jax.experimental import pallas as pl
from jax.experimental.pallas import tpu as pltpu


def _body(m_ref, md_ref, r_ref, rd_ref, s_ref, mo_ref, ro_ref, so_ref):
    mo_ref[...] = m_ref[...] + md_ref[...]
    ro_ref[...] = r_ref[...] + rd_ref[...]
    so_ref[...] = s_ref[...]


def kernel(means, scales, rot, means_def, rot_def):
    n = means.shape[0]
    B = 131072
    g = pl.cdiv(n, B)
    bs3 = pl.BlockSpec((3, B), lambda i: (0, i))
    bs4 = pl.BlockSpec((4, B), lambda i: (0, i))
    mo_t, ro_t, so_t = pl.pallas_call(
        _body,
        grid=(g,),
        in_specs=[bs3, bs3, bs4, bs4, bs3],
        out_specs=[bs3, bs4, bs3],
        out_shape=[
            jax.ShapeDtypeStruct((3, n), means.dtype),
            jax.ShapeDtypeStruct((4, n), rot.dtype),
            jax.ShapeDtypeStruct((3, n), scales.dtype),
        ],
        compiler_params=pltpu.CompilerParams(vmem_limit_bytes=56 * 1024 * 1024),
    )(means.T, means_def.T, rot.T, rot_def.T, scales.T)
    return (mo_t.T, so_t.T, ro_t.T)
